# baseline (speedup 1.0000x reference)
import functools

import jax
import jax.numpy as jnp
from jax import lax
from jax.experimental import pallas as pl
from jax.experimental.pallas import tpu as pltpu
from jax.experimental.pallas import tpu_sc as plsc

N, C, H, W = 2, 192, 128, 128
PH, PW = 7, 7
GH = GW = 2
K = 512
BINS = K * PH * PW
PAIRS = 16
SCALE = 0.25
NC, NS = 2, 16
NW = NC * NS
BPT = BINS // NW
G = 8
NG = BPT // G
CV = C // 16


def _tc_index_body(rois_ref, idx_ref, w_ref):
    r = rois_ref[...]
    b = r[:, 0:1].astype(jnp.int32)
    sw = r[:, 1:2] * SCALE - 0.5
    sh = r[:, 2:3] * SCALE - 0.5
    ew = r[:, 3:4] * SCALE - 0.5
    eh = r[:, 4:5] * SCALE - 0.5
    bin_w = (ew - sw) / PW
    bin_h = (eh - sh) / PH

    j2 = lax.broadcasted_iota(jnp.int32, (K, PAIRS * PH * PW), 1)
    binj = j2 >> 4
    corner = j2 & 3
    sidx = (j2 >> 2) & 3
    iy = (sidx >> 1).astype(jnp.float32)
    ix = (sidx & 1).astype(jnp.float32)
    ph = binj // PW
    pw = binj - ph * PW
    yf = sh + ph.astype(jnp.float32) * bin_h + (iy + 0.5) * bin_h * (1.0 / GH)
    xf = sw + pw.astype(jnp.float32) * bin_w + (ix + 0.5) * bin_w * (1.0 / GW)
    valid = (yf >= -1.0) & (yf <= H) & (xf >= -1.0) & (xf <= W)
    yc = jnp.maximum(yf, 0.0)
    xc = jnp.maximum(xf, 0.0)
    y_low = jnp.minimum(yc.astype(jnp.int32), H - 1)
    x_low = jnp.minimum(xc.astype(jnp.int32), W - 1)
    y_high = jnp.minimum(y_low + 1, H - 1)
    x_high = jnp.minimum(x_low + 1, W - 1)
    yc = jnp.where(y_low >= H - 1, y_low.astype(jnp.float32), yc)
    xc = jnp.where(x_low >= W - 1, x_low.astype(jnp.float32), xc)
    ly = yc - y_low.astype(jnp.float32)
    lx = xc - x_low.astype(jnp.float32)
    wy = jnp.where(corner < 2, 1.0 - ly, ly)
    wx = jnp.where((corner & 1) == 0, 1.0 - lx, lx)
    w = wy * wx * valid.astype(jnp.float32) * (1.0 / (GH * GW))
    rowsel = jnp.where(corner < 2, y_low, y_high)
    colsel = jnp.where((corner & 1) == 0, x_low, x_high)
    idx_ref[...] = b * (H * W) + rowsel * W + colsel
    w_ref[...] = w


def _tc_indices(rois):
    return pl.pallas_call(
        _tc_index_body,
        out_shape=(
            jax.ShapeDtypeStruct((K, PAIRS * PH * PW), jnp.int32),
            jax.ShapeDtypeStruct((K, PAIRS * PH * PW), jnp.float32),
        ),
    )(rois)


def _lane_splat(v, lane):
    return lax.gather(
        v,
        jnp.full((16, 1), lane, jnp.int32),
        lax.GatherDimensionNumbers(
            offset_dims=(), collapsed_slice_dims=(0,), start_index_map=(0,)
        ),
        (1,),
        mode=lax.GatherScatterMode.PROMISE_IN_BOUNDS,
    )


def _sc_body(
    table, idx_hbm, w_hbm, out,
    idx_v0, idx_v1, w_v0, w_v1, rows_v0, rows_v1, out_v, sem0, sem1,
):
    wid = lax.axis_index("s") * NC + lax.axis_index("c")
    bin0 = wid * BPT
    idx_v = (idx_v0, idx_v1)
    w_v = (w_v0, w_v1)
    rows_v = (rows_v0, rows_v1)
    sem = (sem0, sem1)

    def issue(g, p):
        chunk0 = (bin0 + g * G) * PAIRS
        pltpu.sync_copy(idx_hbm.at[pl.ds(chunk0, G * PAIRS)], idx_v[p])
        pltpu.async_copy(table.at[idx_v[p]], rows_v[p], sem[p])
        pltpu.sync_copy(w_hbm.at[pl.ds(chunk0, G * PAIRS)], w_v[p])

    def compute(g, p):
        pltpu.make_async_copy(table.at[idx_v[p]], rows_v[p], sem[p]).wait()

        @plsc.parallel_loop(0, G)
        def _bin(b):
            offs = b * PAIRS
            wall = w_v[p][pl.ds(offs, 16)]
            accs = [jnp.zeros((16,), jnp.float32) for _ in range(CV)]
            for r in range(PAIRS):
                wvec = _lane_splat(wall, r)
                for col in range(CV):
                    chunk = rows_v[p][offs + r, pl.ds(col * 16, 16)]
                    accs[col] = accs[col] + wvec * chunk
            for col in range(CV):
                out_v[b, pl.ds(col * 16, 16)] = accs[col]

        pltpu.sync_copy(out_v, out.at[pl.ds(bin0 + g * G, G)])

    issue(0, 0)

    @pl.loop(0, NG, step=2)
    def _group(g):
        issue(g + 1, 1)
        compute(g, 0)

        @pl.when(g + 2 < NG)
        def _():
            issue(g + 2, 0)

        compute(g + 1, 1)


@jax.jit
def _sc_gather(table, idx_flat, w_flat):
    mesh = plsc.VectorSubcoreMesh(
        core_axis_name="c", subcore_axis_name="s", num_cores=NC, num_subcores=NS
    )
    return pl.kernel(
        _sc_body,
        out_type=jax.ShapeDtypeStruct((BINS, C), jnp.float32),
        mesh=mesh,
        scratch_types=[
            pltpu.VMEM((G * PAIRS,), jnp.int32),
            pltpu.VMEM((G * PAIRS,), jnp.int32),
            pltpu.VMEM((G * PAIRS,), jnp.float32),
            pltpu.VMEM((G * PAIRS,), jnp.float32),
            pltpu.VMEM((G * PAIRS, C), jnp.float32),
            pltpu.VMEM((G * PAIRS, C), jnp.float32),
            pltpu.VMEM((G, C), jnp.float32),
            pltpu.SemaphoreType.DMA,
            pltpu.SemaphoreType.DMA,
        ],
        compiler_params=pltpu.CompilerParams(
            needs_layout_passes=False, use_tc_tiling_on_sc=False
        ),
    )(table, idx_flat, w_flat)


def kernel(input, rois):
    table = input.transpose(0, 2, 3, 1).reshape(N * H * W, C)
    idx2, w2 = _tc_indices(rois)
    out = _sc_gather(table, idx2.reshape(-1), w2.reshape(-1))
    return out.reshape(K, PH * PW, C).transpose(0, 2, 1).reshape(K, C, PH, PW)


# speedup vs baseline: 1.5022x; 1.5022x over previous
import functools

import jax
import jax.numpy as jnp
from jax import lax
from jax.experimental import pallas as pl
from jax.experimental.pallas import tpu as pltpu
from jax.experimental.pallas import tpu_sc as plsc

N, C, H, W = 2, 192, 128, 128
PH, PW = 7, 7
GH = GW = 2
K = 512
BINS = K * PH * PW
PAIRS = 16
SCALE = 0.25
NC, NS = 2, 16
NW = NC * NS
BPT = BINS // NW
G = 8
NG = BPT // G
CV = C // 16


def _tc_index_body(rois_ref, idx_ref, w_ref):
    r = rois_ref[...]
    b = r[:, 0:1].astype(jnp.int32)
    sw = r[:, 1:2] * SCALE - 0.5
    sh = r[:, 2:3] * SCALE - 0.5
    ew = r[:, 3:4] * SCALE - 0.5
    eh = r[:, 4:5] * SCALE - 0.5
    bin_w = (ew - sw) / PW
    bin_h = (eh - sh) / PH

    j2 = lax.broadcasted_iota(jnp.int32, (K, PAIRS * PH * PW), 1)
    binj = j2 >> 4
    corner = j2 & 3
    sidx = (j2 >> 2) & 3
    iy = (sidx >> 1).astype(jnp.float32)
    ix = (sidx & 1).astype(jnp.float32)
    ph = binj // PW
    pw = binj - ph * PW
    yf = sh + ph.astype(jnp.float32) * bin_h + (iy + 0.5) * bin_h * (1.0 / GH)
    xf = sw + pw.astype(jnp.float32) * bin_w + (ix + 0.5) * bin_w * (1.0 / GW)
    valid = (yf >= -1.0) & (yf <= H) & (xf >= -1.0) & (xf <= W)
    yc = jnp.maximum(yf, 0.0)
    xc = jnp.maximum(xf, 0.0)
    y_low = jnp.minimum(yc.astype(jnp.int32), H - 1)
    x_low = jnp.minimum(xc.astype(jnp.int32), W - 1)
    y_high = jnp.minimum(y_low + 1, H - 1)
    x_high = jnp.minimum(x_low + 1, W - 1)
    yc = jnp.where(y_low >= H - 1, y_low.astype(jnp.float32), yc)
    xc = jnp.where(x_low >= W - 1, x_low.astype(jnp.float32), xc)
    ly = yc - y_low.astype(jnp.float32)
    lx = xc - x_low.astype(jnp.float32)
    wy = jnp.where(corner < 2, 1.0 - ly, ly)
    wx = jnp.where((corner & 1) == 0, 1.0 - lx, lx)
    w = wy * wx * valid.astype(jnp.float32) * (1.0 / (GH * GW))
    rowsel = jnp.where(corner < 2, y_low, y_high)
    colsel = jnp.where((corner & 1) == 0, x_low, x_high)
    idx_ref[...] = b * (H * W) + rowsel * W + colsel
    w_ref[...] = w


def _tc_indices(rois):
    return pl.pallas_call(
        _tc_index_body,
        out_shape=(
            jax.ShapeDtypeStruct((K, PAIRS * PH * PW), jnp.int32),
            jax.ShapeDtypeStruct((K, PAIRS * PH * PW), jnp.float32),
        ),
    )(rois)


def _lane_splat(v, lane):
    return lax.gather(
        v,
        jnp.full((16, 1), lane, jnp.int32),
        lax.GatherDimensionNumbers(
            offset_dims=(), collapsed_slice_dims=(0,), start_index_map=(0,)
        ),
        (1,),
        mode=lax.GatherScatterMode.PROMISE_IN_BOUNDS,
    )


def _sc_body(
    table, idx_hbm, w_hbm, out,
    idx_v0, idx_v1, w_v0, w_v1, rows_v0, rows_v1, out_v, sem0, sem1,
):
    wid = lax.axis_index("s") * NC + lax.axis_index("c")
    bin0 = wid * BPT
    idx_v = (idx_v0, idx_v1)
    w_v = (w_v0, w_v1)
    rows_v = (rows_v0, rows_v1)
    sem = (sem0, sem1)

    def issue(g, p):
        chunk0 = (bin0 + g * G) * PAIRS
        pltpu.sync_copy(idx_hbm.at[pl.ds(chunk0, G * PAIRS)], idx_v[p])
        pltpu.async_copy(table.at[idx_v[p]], rows_v[p], sem[p])
        pltpu.sync_copy(w_hbm.at[pl.ds(chunk0, G * PAIRS)], w_v[p])

    def compute(g, p):
        pltpu.make_async_copy(table.at[idx_v[p]], rows_v[p], sem[p]).wait()

        @pl.loop(0, G)
        def _bin(b):
            offs = b * PAIRS
            wall = w_v[p][pl.ds(offs, 16)]
            accs = [jnp.zeros((16,), jnp.float32) for _ in range(CV)]
            for r in range(PAIRS):
                wvec = _lane_splat(wall, r)
                for col in range(CV):
                    chunk = rows_v[p][offs + r, pl.ds(col * 16, 16)]
                    accs[col] = accs[col] + wvec * chunk
            for col in range(CV):
                out_v[b, pl.ds(col * 16, 16)] = accs[col]

        pltpu.sync_copy(out_v, out.at[pl.ds(bin0 + g * G, G)])

    issue(0, 0)

    @pl.loop(0, NG, step=2)
    def _group(g):
        issue(g + 1, 1)
        compute(g, 0)

        @pl.when(g + 2 < NG)
        def _():
            issue(g + 2, 0)

        compute(g + 1, 1)


@jax.jit
def _sc_gather(table, idx_flat, w_flat):
    mesh = plsc.VectorSubcoreMesh(
        core_axis_name="c", subcore_axis_name="s", num_cores=NC, num_subcores=NS
    )
    return pl.kernel(
        _sc_body,
        out_type=jax.ShapeDtypeStruct((BINS, C), jnp.float32),
        mesh=mesh,
        scratch_types=[
            pltpu.VMEM((G * PAIRS,), jnp.int32),
            pltpu.VMEM((G * PAIRS,), jnp.int32),
            pltpu.VMEM((G * PAIRS,), jnp.float32),
            pltpu.VMEM((G * PAIRS,), jnp.float32),
            pltpu.VMEM((G * PAIRS, C), jnp.float32),
            pltpu.VMEM((G * PAIRS, C), jnp.float32),
            pltpu.VMEM((G, C), jnp.float32),
            pltpu.SemaphoreType.DMA,
            pltpu.SemaphoreType.DMA,
        ],
        compiler_params=pltpu.CompilerParams(
            needs_layout_passes=False, use_tc_tiling_on_sc=False
        ),
    )(table, idx_flat, w_flat)


def kernel(input, rois):
    table = input.transpose(0, 2, 3, 1).reshape(N * H * W, C)
    idx2, w2 = _tc_indices(rois)
    out = _sc_gather(table, idx2.reshape(-1), w2.reshape(-1))
    return out.reshape(K, PH * PW, C).transpose(0, 2, 1).reshape(K, C, PH, PW)


# speedup vs baseline: 1.5955x; 1.0621x over previous
import functools

import jax
import jax.numpy as jnp
from jax import lax
from jax.experimental import pallas as pl
from jax.experimental.pallas import tpu as pltpu
from jax.experimental.pallas import tpu_sc as plsc

N, C, H, W = 2, 192, 128, 128
PH, PW = 7, 7
GH = GW = 2
K = 512
BINS = K * PH * PW
PAIRS = 16
SCALE = 0.25
NC, NS = 2, 16
NW = NC * NS
BPT = BINS // NW
G = 8
NG = BPT // G
CV = C // 16
CB = C // 32


def _tc_index_body(rois_ref, idx_ref, w_ref):
    r = rois_ref[...]
    b = r[:, 0:1].astype(jnp.int32)
    sw = r[:, 1:2] * SCALE - 0.5
    sh = r[:, 2:3] * SCALE - 0.5
    ew = r[:, 3:4] * SCALE - 0.5
    eh = r[:, 4:5] * SCALE - 0.5
    bin_w = (ew - sw) / PW
    bin_h = (eh - sh) / PH

    j2 = lax.broadcasted_iota(jnp.int32, (K, PAIRS * PH * PW), 1)
    binj = j2 >> 4
    corner = j2 & 3
    sidx = (j2 >> 2) & 3
    iy = (sidx >> 1).astype(jnp.float32)
    ix = (sidx & 1).astype(jnp.float32)
    ph = binj // PW
    pw = binj - ph * PW
    yf = sh + ph.astype(jnp.float32) * bin_h + (iy + 0.5) * bin_h * (1.0 / GH)
    xf = sw + pw.astype(jnp.float32) * bin_w + (ix + 0.5) * bin_w * (1.0 / GW)
    valid = (yf >= -1.0) & (yf <= H) & (xf >= -1.0) & (xf <= W)
    yc = jnp.maximum(yf, 0.0)
    xc = jnp.maximum(xf, 0.0)
    y_low = jnp.minimum(yc.astype(jnp.int32), H - 1)
    x_low = jnp.minimum(xc.astype(jnp.int32), W - 1)
    y_high = jnp.minimum(y_low + 1, H - 1)
    x_high = jnp.minimum(x_low + 1, W - 1)
    yc = jnp.where(y_low >= H - 1, y_low.astype(jnp.float32), yc)
    xc = jnp.where(x_low >= W - 1, x_low.astype(jnp.float32), xc)
    ly = yc - y_low.astype(jnp.float32)
    lx = xc - x_low.astype(jnp.float32)
    wy = jnp.where(corner < 2, 1.0 - ly, ly)
    wx = jnp.where((corner & 1) == 0, 1.0 - lx, lx)
    w = wy * wx * valid.astype(jnp.float32) * (1.0 / (GH * GW))
    rowsel = jnp.where(corner < 2, y_low, y_high)
    colsel = jnp.where((corner & 1) == 0, x_low, x_high)
    idx_ref[...] = b * (H * W) + rowsel * W + colsel
    wu = lax.bitcast_convert_type(w.astype(jnp.bfloat16), jnp.uint16)
    wu = wu.astype(jnp.uint32)
    w_ref[...] = (wu | (wu << 16)).astype(jnp.int32)


def _tc_indices(rois):
    return pl.pallas_call(
        _tc_index_body,
        out_shape=(
            jax.ShapeDtypeStruct((K, PAIRS * PH * PW), jnp.int32),
            jax.ShapeDtypeStruct((K, PAIRS * PH * PW), jnp.int32),
        ),
    )(rois)


def _lane_splat(v, lane):
    return lax.gather(
        v,
        jnp.full((16, 1), lane, jnp.int32),
        lax.GatherDimensionNumbers(
            offset_dims=(), collapsed_slice_dims=(0,), start_index_map=(0,)
        ),
        (1,),
        mode=lax.GatherScatterMode.PROMISE_IN_BOUNDS,
    )


def _sc_body(
    table, idx_hbm, w_hbm, out,
    idx_v0, idx_v1, w_v0, w_v1, rows_v0, rows_v1, out_v, sem0, sem1,
):
    wid = lax.axis_index("s") * NC + lax.axis_index("c")
    bin0 = wid * BPT
    idx_v = (idx_v0, idx_v1)
    w_v = (w_v0, w_v1)
    rows_v = (rows_v0, rows_v1)
    sem = (sem0, sem1)

    def issue(g, p):
        chunk0 = (bin0 + g * G) * PAIRS
        pltpu.sync_copy(idx_hbm.at[pl.ds(chunk0, G * PAIRS)], idx_v[p])
        pltpu.async_copy(table.at[idx_v[p]], rows_v[p], sem[p])
        pltpu.sync_copy(w_hbm.at[pl.ds(chunk0, G * PAIRS)], w_v[p])

    def compute(g, p):
        pltpu.make_async_copy(table.at[idx_v[p]], rows_v[p], sem[p]).wait()

        lanes2 = 2 * lax.broadcasted_iota(jnp.int32, (16,), 0)

        @pl.loop(0, G)
        def _bin(b):
            offs = b * PAIRS
            wall = w_v[p][pl.ds(offs, 16)]
            acc_e = [jnp.zeros((16,), jnp.float32) for _ in range(CB)]
            acc_o = [jnp.zeros((16,), jnp.float32) for _ in range(CB)]
            parts = [None] * CB
            for r in range(PAIRS):
                wspl = plsc.bitcast(_lane_splat(wall, r), jnp.bfloat16)
                for col in range(CB):
                    chunk = rows_v[p][offs + r, pl.ds(col * 32, 32)]
                    prod = chunk * wspl
                    parts[col] = prod if r % 4 == 0 else parts[col] + prod
                if r % 4 == 3:
                    for col in range(CB):
                        lo, hi = plsc.unpack(
                            parts[col], format=plsc.PackFormat.INTERLEAVED
                        )
                        acc_e[col] = acc_e[col] + lo
                        acc_o[col] = acc_o[col] + hi
            brow = jnp.full((16,), b, jnp.int32)
            for col in range(CB):
                plsc.store_scatter(out_v, [brow, col * 32 + lanes2], acc_e[col])
                plsc.store_scatter(out_v, [brow, col * 32 + lanes2 + 1], acc_o[col])

        pltpu.sync_copy(out_v, out.at[pl.ds(bin0 + g * G, G)])

    issue(0, 0)

    @pl.loop(0, NG, step=2)
    def _group(g):
        issue(g + 1, 1)
        compute(g, 0)

        @pl.when(g + 2 < NG)
        def _():
            issue(g + 2, 0)

        compute(g + 1, 1)


@jax.jit
def _sc_gather(table, idx_flat, w_flat):
    mesh = plsc.VectorSubcoreMesh(
        core_axis_name="c", subcore_axis_name="s", num_cores=NC, num_subcores=NS
    )
    return pl.kernel(
        _sc_body,
        out_type=jax.ShapeDtypeStruct((BINS, C), jnp.float32),
        mesh=mesh,
        scratch_types=[
            pltpu.VMEM((G * PAIRS,), jnp.int32),
            pltpu.VMEM((G * PAIRS,), jnp.int32),
            pltpu.VMEM((G * PAIRS,), jnp.int32),
            pltpu.VMEM((G * PAIRS,), jnp.int32),
            pltpu.VMEM((G * PAIRS, C), jnp.bfloat16),
            pltpu.VMEM((G * PAIRS, C), jnp.bfloat16),
            pltpu.VMEM((G, C), jnp.float32),
            pltpu.SemaphoreType.DMA,
            pltpu.SemaphoreType.DMA,
        ],
        compiler_params=pltpu.CompilerParams(
            needs_layout_passes=False, use_tc_tiling_on_sc=False
        ),
    )(table, idx_flat, w_flat)


def kernel(input, rois):
    table = input.transpose(0, 2, 3, 1).reshape(N * H * W, C).astype(jnp.bfloat16)
    idx2, w2 = _tc_indices(rois)
    out = _sc_gather(table, idx2.reshape(-1), w2.reshape(-1))
    return out.reshape(K, PH * PW, C).transpose(0, 2, 1).reshape(K, C, PH, PW)
